# baseline (speedup 1.0000x reference)
import jax
import jax.numpy as jnp
from jax import lax
from jax.experimental import pallas as pl
from jax.experimental.pallas import tpu as pltpu
from jax.experimental.pallas import tpu_sc as plsc

N = 100000
NUM_FIELDS = 9
VOCAB = 100
HIDDEN = 128
NCH = HIDDEN // 16

NW = 32
BB = 80
NBLK = N // BB
BLK_PER_W = -(-NBLK // NW)


def _body(idx_hbm, ft_hbm, out_hbm,
          idx0, idx1, acc0, acc1, s0, s1,
          semA0, semA1, semB0, semB1, semC0, semC1, semI0, semI1):
    wid = lax.axis_index("s") * 2 + lax.axis_index("c")
    idx = (idx0, idx1)
    acc = (acc0, acc1)
    stg = (s0, s1)
    semA = (semA0, semA1)
    semB = (semB0, semB1)
    semC = (semC0, semC1)
    semI = (semI0, semI1)

    def wait_rows(dst, sem):
        pltpu.make_async_copy(ft_hbm.at[idx0.at[0]], dst, sem).wait()

    def add_field(acc_ref, stg_ref):
        @plsc.parallel_loop(0, BB, unroll=4)
        def _(r):
            for c in range(NCH):
                sl = pl.ds(c * 16, 16)
                plsc.addupdate(acc_ref.at[r, sl], stg_ref[r, sl])

    def block(k, p):
        blk = k * NW + wid

        @pl.when(blk < NBLK)
        def _():
            nxt = blk + NW
            has_next = nxt < NBLK
            q = 1 - p

            @pl.when(has_next)
            def _():
                pltpu.async_copy(idx_hbm.at[nxt], idx[q], semI[q])

            wait_rows(acc[p], semA[p])
            for j in range(1, NUM_FIELDS - 1):
                pltpu.async_copy(
                    ft_hbm.at[idx[p].at[j + 1]], stg[(j + 1) % 2],
                    semB[(j + 1) % 2])
                wait_rows(stg[j % 2], semB[j % 2])
                add_field(acc[p], stg[j % 2])

            @pl.when(has_next)
            def _():
                pltpu.make_async_copy(idx_hbm.at[0], idx[q], semI[q]).wait()

                @pl.when(k >= 1)
                def _():
                    wait_rows(acc[q], semC[q])

                pltpu.async_copy(ft_hbm.at[idx[q].at[0]], acc[q], semA[q])
                pltpu.async_copy(ft_hbm.at[idx[q].at[1]], stg[1], semB[1])

            wait_rows(stg[0], semB[0])
            add_field(acc[p], stg[0])
            pltpu.async_copy(acc[p], out_hbm.at[pl.ds(blk * BB, BB)], semC[p])

    pltpu.sync_copy(idx_hbm.at[wid], idx0)
    pltpu.async_copy(ft_hbm.at[idx0.at[0]], acc0, semA0)
    pltpu.async_copy(ft_hbm.at[idx0.at[1]], s1, semB1)

    def pair(kk, _):
        block(2 * kk, 0)
        block(2 * kk + 1, 1)
        return 0

    lax.fori_loop(0, BLK_PER_W // 2, pair, 0)

    wait_rows(acc0, semC0)
    wait_rows(acc1, semC1)


@jax.jit
def kernel(x, tables):
    ft = tables.reshape(NUM_FIELDS * VOCAB, HIDDEN)
    offs = (jnp.arange(NUM_FIELDS, dtype=jnp.int32) * VOCAB)[:, None]
    idxT = x.astype(jnp.int32).T + offs
    idx3 = idxT.reshape(NUM_FIELDS, NBLK, BB).transpose(1, 0, 2)

    mesh = plsc.VectorSubcoreMesh(core_axis_name="c", subcore_axis_name="s")
    run = pl.kernel(
        _body,
        out_type=jax.ShapeDtypeStruct((N, HIDDEN), jnp.float32),
        mesh=mesh,
        scratch_types=[
            pltpu.VMEM((NUM_FIELDS, BB), jnp.int32),
            pltpu.VMEM((NUM_FIELDS, BB), jnp.int32),
            pltpu.VMEM((BB, HIDDEN), jnp.float32),
            pltpu.VMEM((BB, HIDDEN), jnp.float32),
            pltpu.VMEM((BB, HIDDEN), jnp.float32),
            pltpu.VMEM((BB, HIDDEN), jnp.float32),
            pltpu.SemaphoreType.DMA,
            pltpu.SemaphoreType.DMA,
            pltpu.SemaphoreType.DMA,
            pltpu.SemaphoreType.DMA,
            pltpu.SemaphoreType.DMA,
            pltpu.SemaphoreType.DMA,
            pltpu.SemaphoreType.DMA,
            pltpu.SemaphoreType.DMA,
        ],
    )
    return run(idx3, ft)


# speedup vs baseline: 1.4951x; 1.4951x over previous
import jax
import jax.numpy as jnp
from jax import lax
from jax.experimental import pallas as pl
from jax.experimental.pallas import tpu as pltpu
from jax.experimental.pallas import tpu_sc as plsc

N = 100000
NUM_FIELDS = 9
VOCAB = 100
HIDDEN = 128
NCH = HIDDEN // 16

NPAIR = 4
NLOOK = NPAIR + 1

NW = 32
BB = 80
NBLK = N // BB
BLK_PER_W = -(-NBLK // NW)


def _pair_body(te_ref, to_ref, out_ref):
    out_ref[0] = te_ref[0][:, None, :] + to_ref[0][None, :, :]


def _build_pair_tables(tables):
    grid = (NPAIR,)
    return pl.pallas_call(
        _pair_body,
        grid=grid,
        in_specs=[
            pl.BlockSpec((1, VOCAB, HIDDEN), lambda k: (2 * k, 0, 0)),
            pl.BlockSpec((1, VOCAB, HIDDEN), lambda k: (2 * k + 1, 0, 0)),
        ],
        out_specs=pl.BlockSpec((1, VOCAB, VOCAB, HIDDEN),
                               lambda k: (k, 0, 0, 0)),
        out_shape=jax.ShapeDtypeStruct((NPAIR, VOCAB, VOCAB, HIDDEN),
                                       jnp.float32),
    )(tables, tables)


def _body(idx_hbm, big_hbm, ft_hbm, out_hbm,
          idx0, idx1, acc0, acc1, s0, s1,
          semA0, semA1, semB0, semB1, semC0, semC1, semI0, semI1):
    wid = lax.axis_index("s") * 2 + lax.axis_index("c")
    idx = (idx0, idx1)
    acc = (acc0, acc1)
    stg = (s0, s1)
    semA = (semA0, semA1)
    semB = (semB0, semB1)
    semC = (semC0, semC1)
    semI = (semI0, semI1)

    def src(j):
        return big_hbm if j < NPAIR else ft_hbm

    def wait_rows(dst, sem):
        pltpu.make_async_copy(ft_hbm.at[idx0.at[0]], dst, sem).wait()

    def add_field(acc_ref, stg_ref):
        @plsc.parallel_loop(0, BB, unroll=2)
        def _(r):
            for c in range(NCH):
                sl = pl.ds(c * 16, 16)
                plsc.addupdate(acc_ref.at[r, sl], stg_ref[r, sl])

    def block(k, p):
        blk = k * NW + wid

        @pl.when(blk < NBLK)
        def _():
            nxt = blk + NW
            has_next = nxt < NBLK
            q = 1 - p

            @pl.when(has_next)
            def _():
                pltpu.async_copy(idx_hbm.at[nxt], idx[q], semI[q])

            wait_rows(acc[p], semA[p])
            for j in range(1, NLOOK - 1):
                pltpu.async_copy(
                    src(j + 1).at[idx[p].at[j + 1]], stg[(j + 1) % 2],
                    semB[(j + 1) % 2])
                wait_rows(stg[j % 2], semB[j % 2])
                add_field(acc[p], stg[j % 2])

            @pl.when(has_next)
            def _():
                pltpu.make_async_copy(idx_hbm.at[0], idx[q], semI[q]).wait()

                @pl.when(k >= 1)
                def _():
                    wait_rows(acc[q], semC[q])

                pltpu.async_copy(big_hbm.at[idx[q].at[0]], acc[q], semA[q])
                pltpu.async_copy(big_hbm.at[idx[q].at[1]], stg[1], semB[1])

            wait_rows(stg[0], semB[0])
            add_field(acc[p], stg[0])
            pltpu.async_copy(acc[p], out_hbm.at[pl.ds(blk * BB, BB)], semC[p])

    pltpu.sync_copy(idx_hbm.at[wid], idx0)
    pltpu.async_copy(big_hbm.at[idx0.at[0]], acc0, semA0)
    pltpu.async_copy(big_hbm.at[idx0.at[1]], s1, semB1)

    def pair(kk, _):
        block(2 * kk, 0)
        block(2 * kk + 1, 1)
        return 0

    lax.fori_loop(0, BLK_PER_W // 2, pair, 0)

    wait_rows(acc0, semC0)
    wait_rows(acc1, semC1)


@jax.jit
def kernel(x, tables):
    xi = x.astype(jnp.int32)
    ft = tables.reshape(NUM_FIELDS * VOCAB, HIDDEN)
    big = _build_pair_tables(tables).reshape(NPAIR * VOCAB * VOCAB, HIDDEN)

    poff = (jnp.arange(NPAIR, dtype=jnp.int32) * VOCAB * VOCAB)[None, :]
    pidx = xi[:, 0:2 * NPAIR:2] * VOCAB + xi[:, 1:2 * NPAIR:2] + poff
    f8 = xi[:, 8:9] + (NUM_FIELDS - 1) * VOCAB
    idxT = jnp.concatenate([pidx, f8], axis=1).T
    idx3 = idxT.reshape(NLOOK, NBLK, BB).transpose(1, 0, 2)

    mesh = plsc.VectorSubcoreMesh(core_axis_name="c", subcore_axis_name="s")
    run = pl.kernel(
        _body,
        out_type=jax.ShapeDtypeStruct((N, HIDDEN), jnp.float32),
        mesh=mesh,
        scratch_types=[
            pltpu.VMEM((NLOOK, BB), jnp.int32),
            pltpu.VMEM((NLOOK, BB), jnp.int32),
            pltpu.VMEM((BB, HIDDEN), jnp.float32),
            pltpu.VMEM((BB, HIDDEN), jnp.float32),
            pltpu.VMEM((BB, HIDDEN), jnp.float32),
            pltpu.VMEM((BB, HIDDEN), jnp.float32),
            pltpu.SemaphoreType.DMA,
            pltpu.SemaphoreType.DMA,
            pltpu.SemaphoreType.DMA,
            pltpu.SemaphoreType.DMA,
            pltpu.SemaphoreType.DMA,
            pltpu.SemaphoreType.DMA,
            pltpu.SemaphoreType.DMA,
            pltpu.SemaphoreType.DMA,
        ],
    )
    return run(idx3, big, ft)


# speedup vs baseline: 1.5874x; 1.0617x over previous
import jax
import jax.numpy as jnp
from jax import lax
from jax.experimental import pallas as pl
from jax.experimental.pallas import tpu as pltpu
from jax.experimental.pallas import tpu_sc as plsc

N = 100000
NUM_FIELDS = 9
VOCAB = 100
HIDDEN = 128
NCH = HIDDEN // 16

NPAIR = 4
NLOOK = NPAIR + 1

NW = 32
BB = 80
XPB = BB * NUM_FIELDS
NBLK = N // BB
BLK_PER_W = -(-NBLK // NW)


def _pair_body(te_ref, to_ref, out_ref):
    out_ref[0] = te_ref[0][:, None, :] + to_ref[0][None, :, :]


def _build_pair_tables(tables):
    return pl.pallas_call(
        _pair_body,
        grid=(NPAIR,),
        in_specs=[
            pl.BlockSpec((1, VOCAB, HIDDEN), lambda k: (2 * k, 0, 0)),
            pl.BlockSpec((1, VOCAB, HIDDEN), lambda k: (2 * k + 1, 0, 0)),
        ],
        out_specs=pl.BlockSpec((1, VOCAB, VOCAB, HIDDEN),
                               lambda k: (k, 0, 0, 0)),
        out_shape=jax.ShapeDtypeStruct((NPAIR, VOCAB, VOCAB, HIDDEN),
                                       jnp.float32),
    )(tables, tables)


def _body(x_hbm, big_hbm, ft_hbm, out_hbm,
          xv0, xv1, lk0, lk1, acc0, acc1, s0, s1,
          semA0, semA1, semB0, semB1, semC0, semC1, semI0, semI1):
    wid = lax.axis_index("s") * 2 + lax.axis_index("c")
    xv = (xv0, xv1)
    lk = (lk0, lk1)
    acc = (acc0, acc1)
    stg = (s0, s1)
    semA = (semA0, semA1)
    semB = (semB0, semB1)
    semC = (semC0, semC1)
    semI = (semI0, semI1)

    def src(j):
        return big_hbm if j < NPAIR else ft_hbm

    def wait_rows(dst, sem):
        pltpu.make_async_copy(ft_hbm.at[lk0.at[0]], dst, sem).wait()

    def compute_lk(xv_ref, lk_ref):
        for rc in range(BB // 16):
            sl = pl.ds(rc * 16, 16)
            for j in range(NPAIR):
                lk_ref[j, sl] = (xv_ref[2 * j, sl] * VOCAB
                                 + xv_ref[2 * j + 1, sl]
                                 + j * (VOCAB * VOCAB))
            lk_ref[NPAIR, sl] = xv_ref[NUM_FIELDS - 1, sl] + (
                (NUM_FIELDS - 1) * VOCAB)

    def add_field(acc_ref, stg_ref):
        @plsc.parallel_loop(0, BB, unroll=2)
        def _(r):
            for c in range(NCH):
                sl = pl.ds(c * 16, 16)
                plsc.addupdate(acc_ref.at[r, sl], stg_ref[r, sl])

    def block(k, p):
        blk = k * NW + wid

        @pl.when(blk < NBLK)
        def _():
            nxt = blk + NW
            has_next = nxt < NBLK
            q = 1 - p

            @pl.when(has_next)
            def _():
                for j in range(NUM_FIELDS):
                    pltpu.async_copy(x_hbm.at[pl.ds(j * N + nxt * BB, BB)],
                                     xv[q].at[j], semI[q])

            wait_rows(acc[p], semA[p])
            for j in range(1, NLOOK - 1):
                pltpu.async_copy(
                    src(j + 1).at[lk[p].at[j + 1]], stg[(j + 1) % 2],
                    semB[(j + 1) % 2])
                wait_rows(stg[j % 2], semB[j % 2])
                add_field(acc[p], stg[j % 2])

            @pl.when(has_next)
            def _():
                for j in range(NUM_FIELDS):
                    pltpu.make_async_copy(x_hbm.at[pl.ds(0, BB)],
                                          xv[q].at[j], semI[q]).wait()
                compute_lk(xv[q], lk[q])

                @pl.when(k >= 1)
                def _():
                    wait_rows(acc[q], semC[q])

                pltpu.async_copy(big_hbm.at[lk[q].at[0]], acc[q], semA[q])
                pltpu.async_copy(big_hbm.at[lk[q].at[1]], stg[1], semB[1])

            wait_rows(stg[0], semB[0])
            add_field(acc[p], stg[0])
            pltpu.async_copy(acc[p], out_hbm.at[pl.ds(blk * BB, BB)], semC[p])

    for j in range(NUM_FIELDS):
        pltpu.sync_copy(x_hbm.at[pl.ds(j * N + wid * BB, BB)], xv0.at[j])
    compute_lk(xv0, lk0)
    pltpu.async_copy(big_hbm.at[lk0.at[0]], acc0, semA0)
    pltpu.async_copy(big_hbm.at[lk0.at[1]], s1, semB1)

    def pair(kk, _):
        block(2 * kk, 0)
        block(2 * kk + 1, 1)
        return 0

    lax.fori_loop(0, BLK_PER_W // 2, pair, 0)

    wait_rows(acc0, semC0)
    wait_rows(acc1, semC1)


@jax.jit
def kernel(x, tables):
    xflat = x.astype(jnp.int32).T.reshape(NUM_FIELDS * N)
    ft = tables.reshape(NUM_FIELDS * VOCAB, HIDDEN)
    big = _build_pair_tables(tables).reshape(NPAIR * VOCAB * VOCAB, HIDDEN)

    mesh = plsc.VectorSubcoreMesh(core_axis_name="c", subcore_axis_name="s")
    run = pl.kernel(
        _body,
        out_type=jax.ShapeDtypeStruct((N, HIDDEN), jnp.float32),
        mesh=mesh,
        scratch_types=[
            pltpu.VMEM((NUM_FIELDS, BB), jnp.int32),
            pltpu.VMEM((NUM_FIELDS, BB), jnp.int32),
            pltpu.VMEM((NLOOK, BB), jnp.int32),
            pltpu.VMEM((NLOOK, BB), jnp.int32),
            pltpu.VMEM((BB, HIDDEN), jnp.float32),
            pltpu.VMEM((BB, HIDDEN), jnp.float32),
            pltpu.VMEM((BB, HIDDEN), jnp.float32),
            pltpu.VMEM((BB, HIDDEN), jnp.float32),
            pltpu.SemaphoreType.DMA,
            pltpu.SemaphoreType.DMA,
            pltpu.SemaphoreType.DMA,
            pltpu.SemaphoreType.DMA,
            pltpu.SemaphoreType.DMA,
            pltpu.SemaphoreType.DMA,
            pltpu.SemaphoreType.DMA,
            pltpu.SemaphoreType.DMA,
        ],
    )
    return run(xflat, big, ft)


# speedup vs baseline: 1.6099x; 1.0142x over previous
import jax
import jax.numpy as jnp
from jax import lax
from jax.experimental import pallas as pl
from jax.experimental.pallas import tpu as pltpu
from jax.experimental.pallas import tpu_sc as plsc

N = 100000
NUM_FIELDS = 9
VOCAB = 100
HIDDEN = 128
NCH = HIDDEN // 16

NPAIR = 4
NLOOK = NPAIR + 1

NW = 32
BB = 80
NBLK = N // BB
BLK_PER_W = -(-NBLK // NW)


def _pair_body(te_ref, to_ref, out_ref):
    out_ref[0] = te_ref[0][:, None, :] + to_ref[0][None, :, :]


def _build_pair_tables(tables):
    return pl.pallas_call(
        _pair_body,
        grid=(NPAIR,),
        in_specs=[
            pl.BlockSpec((1, VOCAB, HIDDEN), lambda k: (2 * k, 0, 0)),
            pl.BlockSpec((1, VOCAB, HIDDEN), lambda k: (2 * k + 1, 0, 0)),
        ],
        out_specs=pl.BlockSpec((1, VOCAB, VOCAB, HIDDEN),
                               lambda k: (k, 0, 0, 0)),
        out_shape=jax.ShapeDtypeStruct((NPAIR, VOCAB, VOCAB, HIDDEN),
                                       jnp.float32),
    )(tables, tables)


def _body(x_hbm, big_hbm, ft_hbm, out_hbm,
          xv0, xv1, lk0, lk1, acc0, acc1,
          s00, s01, s02, s03, s10, s11, s12, s13,
          semA0, semA1, semB0, semB1, semC0, semC1, semI0, semI1):
    wid = lax.axis_index("s") * 2 + lax.axis_index("c")
    xv = (xv0, xv1)
    lk = (lk0, lk1)
    acc = (acc0, acc1)
    stg = ((s00, s01, s02, s03), (s10, s11, s12, s13))
    semA = (semA0, semA1)
    semB = (semB0, semB1)
    semC = (semC0, semC1)
    semI = (semI0, semI1)

    def wait_rows(dst, sem):
        pltpu.make_async_copy(ft_hbm.at[lk0.at[0]], dst, sem).wait()

    def fetch_x(blk, q):
        for j in range(NUM_FIELDS):
            pltpu.async_copy(x_hbm.at[pl.ds(j * N + blk * BB, BB)],
                             xv[q].at[j], semI[q])

    def compute_lk(xv_ref, lk_ref):
        for rc in range(BB // 16):
            sl = pl.ds(rc * 16, 16)
            for j in range(NPAIR):
                lk_ref[j, sl] = (xv_ref[2 * j, sl] * VOCAB
                                 + xv_ref[2 * j + 1, sl]
                                 + j * (VOCAB * VOCAB))
            lk_ref[NPAIR, sl] = xv_ref[NUM_FIELDS - 1, sl] + (
                (NUM_FIELDS - 1) * VOCAB)

    def fire_gathers(q):
        pltpu.async_copy(big_hbm.at[lk[q].at[0]], acc[q], semA[q])
        for j in range(1, NPAIR):
            pltpu.async_copy(big_hbm.at[lk[q].at[j]], stg[q][j - 1], semB[q])
        pltpu.async_copy(ft_hbm.at[lk[q].at[NPAIR]], stg[q][NPAIR - 1],
                         semB[q])

    def block(k, p):
        blk = k * NW + wid

        @pl.when(blk < NBLK)
        def _():
            nxt = blk + NW
            has_next = nxt < NBLK
            q = 1 - p

            @pl.when(has_next)
            def _():
                fetch_x(nxt, q)

            wait_rows(acc[p], semA[p])
            for _ in range(NLOOK - 1):
                wait_rows(stg[p][0], semB[p])

            @pl.when(has_next)
            def _():
                for j in range(NUM_FIELDS):
                    pltpu.make_async_copy(x_hbm.at[pl.ds(0, BB)],
                                          xv[q].at[j], semI[q]).wait()
                compute_lk(xv[q], lk[q])

                @pl.when(k >= 1)
                def _():
                    wait_rows(acc[q], semC[q])

                fire_gathers(q)

            a = acc[p]
            s0, s1, s2, s3 = stg[p]

            @plsc.parallel_loop(0, BB, unroll=2)
            def _(r):
                for c in range(NCH):
                    sl = pl.ds(c * 16, 16)
                    t01 = s0[r, sl] + s1[r, sl]
                    t23 = s2[r, sl] + s3[r, sl]
                    a[r, sl] = a[r, sl] + (t01 + t23)

            pltpu.async_copy(acc[p], out_hbm.at[pl.ds(blk * BB, BB)], semC[p])

    for j in range(NUM_FIELDS):
        pltpu.sync_copy(x_hbm.at[pl.ds(j * N + wid * BB, BB)], xv0.at[j])
    compute_lk(xv0, lk0)
    fire_gathers(0)

    def pair(kk, _):
        block(2 * kk, 0)
        block(2 * kk + 1, 1)
        return 0

    lax.fori_loop(0, BLK_PER_W // 2, pair, 0)

    wait_rows(acc0, semC0)
    wait_rows(acc1, semC1)


@jax.jit
def kernel(x, tables):
    xflat = x.astype(jnp.int32).T.reshape(NUM_FIELDS * N)
    ft = tables.reshape(NUM_FIELDS * VOCAB, HIDDEN)
    big = _build_pair_tables(tables).reshape(NPAIR * VOCAB * VOCAB, HIDDEN)

    mesh = plsc.VectorSubcoreMesh(core_axis_name="c", subcore_axis_name="s")
    run = pl.kernel(
        _body,
        out_type=jax.ShapeDtypeStruct((N, HIDDEN), jnp.float32),
        mesh=mesh,
        scratch_types=[
            pltpu.VMEM((NUM_FIELDS, BB), jnp.int32),
            pltpu.VMEM((NUM_FIELDS, BB), jnp.int32),
            pltpu.VMEM((NLOOK, BB), jnp.int32),
            pltpu.VMEM((NLOOK, BB), jnp.int32),
            pltpu.VMEM((BB, HIDDEN), jnp.float32),
            pltpu.VMEM((BB, HIDDEN), jnp.float32),
            pltpu.VMEM((BB, HIDDEN), jnp.float32),
            pltpu.VMEM((BB, HIDDEN), jnp.float32),
            pltpu.VMEM((BB, HIDDEN), jnp.float32),
            pltpu.VMEM((BB, HIDDEN), jnp.float32),
            pltpu.VMEM((BB, HIDDEN), jnp.float32),
            pltpu.VMEM((BB, HIDDEN), jnp.float32),
            pltpu.VMEM((BB, HIDDEN), jnp.float32),
            pltpu.VMEM((BB, HIDDEN), jnp.float32),
            pltpu.SemaphoreType.DMA,
            pltpu.SemaphoreType.DMA,
            pltpu.SemaphoreType.DMA,
            pltpu.SemaphoreType.DMA,
            pltpu.SemaphoreType.DMA,
            pltpu.SemaphoreType.DMA,
            pltpu.SemaphoreType.DMA,
            pltpu.SemaphoreType.DMA,
        ],
    )
    return run(xflat, big, ft)
